# Initial kernel scaffold; baseline (speedup 1.0000x reference)
import jax
import jax.numpy as jnp
from jax.experimental import pallas as pl


def kernel(indices, table, W, b):
    raise NotImplementedError("write your pallas kernel here")


# speedup vs baseline: 101.3587x; 101.3587x over previous
import functools

import jax
import jax.numpy as jnp
from jax import lax
from jax.experimental import pallas as pl
from jax.experimental.pallas import tpu as pltpu
from jax.experimental.pallas import tpu_sc as plsc

_NC = 2
_NS = 16
_NW = _NC * _NS
_L = 16


@functools.partial(jax.jit, static_argnames=("n_total", "dim", "chunk"))
def _sc_lookup(idx_flat, params, *, n_total, dim, chunk):
  n_per_w = n_total // _NW
  n_chunks = n_per_w // chunk
  p_rows = params.shape[0]

  mesh = plsc.VectorSubcoreMesh(core_axis_name="c", subcore_axis_name="s")

  @functools.partial(
      pl.kernel,
      mesh=mesh,
      out_type=jax.ShapeDtypeStruct((n_total,), jnp.float32),
      scratch_types=[
          pltpu.VMEM((p_rows, _L), jnp.float32),
          pltpu.VMEM((chunk,), jnp.int32),
          pltpu.VMEM((chunk,), jnp.float32),
      ],
  )
  def k(idx_hbm, params_hbm, out_hbm, params_v, idx_v, out_v):
    wid = lax.axis_index("s") * _NC + lax.axis_index("c")
    base = wid * n_per_w

    pltpu.sync_copy(params_hbm, params_v)
    acc = jnp.zeros((_L,), jnp.float32)
    for d in range(dim):
      acc = acc + params_v[d] * params_v[dim + d]
    s = 1.0 / (1.0 + jnp.exp(-(acc + params_v[2 * dim])))

    for c in range(n_chunks):
      off = base + c * chunk
      pltpu.sync_copy(idx_hbm.at[pl.ds(off, chunk)], idx_v)

      dnums = lax.GatherDimensionNumbers(
          offset_dims=(), collapsed_slice_dims=(0,), start_index_map=(0,))

      def body(i, carry):
        o = i * _L
        iv = idx_v[pl.ds(o, _L)]
        out_v[pl.ds(o, _L)] = lax.gather(
            carry, iv[:, None], dnums, (1,),
            mode=lax.GatherScatterMode.PROMISE_IN_BOUNDS)
        return carry

      lax.fori_loop(0, chunk // _L, body, s)
      pltpu.sync_copy(out_v, out_hbm.at[pl.ds(off, chunk)])

  return k(idx_flat, params)


def kernel(indices, table, W, b):
  n_vocab, dim = table.shape
  out_shape = indices.shape + (1,)
  n_total = indices.size

  idx_flat = indices.reshape(-1).astype(jnp.int32)
  tcols = jnp.zeros((dim, _L), jnp.float32).at[:, :n_vocab].set(
      table.astype(jnp.float32).T)
  wrows = jnp.broadcast_to(W.astype(jnp.float32).reshape(dim, 1), (dim, _L))
  brow = jnp.broadcast_to(b.astype(jnp.float32).reshape(1, 1), (1, _L))
  params = jnp.concatenate([tcols, wrows, brow], axis=0)

  assert n_total % _NW == 0
  n_per_w = n_total // _NW
  chunk = n_per_w
  while chunk * 8 > 420000:
    chunk //= 2
  assert n_per_w % chunk == 0 and chunk % _L == 0

  out_flat = _sc_lookup(idx_flat, params, n_total=n_total, dim=dim, chunk=chunk)
  return out_flat.reshape(out_shape)


# speedup vs baseline: 121.8523x; 1.2022x over previous
import functools

import jax
import jax.numpy as jnp
from jax import lax
from jax.experimental import pallas as pl
from jax.experimental.pallas import tpu as pltpu
from jax.experimental.pallas import tpu_sc as plsc

_NC = 2
_NS = 16
_NW = _NC * _NS
_L = 16


@functools.partial(jax.jit, static_argnames=("n_total", "dim", "chunk"))
def _sc_lookup(idx_flat, params, *, n_total, dim, chunk):
  n_per_w = n_total // _NW
  n_chunks = n_per_w // chunk
  p_rows = params.shape[0]

  mesh = plsc.VectorSubcoreMesh(core_axis_name="c", subcore_axis_name="s")

  @functools.partial(
      pl.kernel,
      mesh=mesh,
      out_type=jax.ShapeDtypeStruct((n_total,), jnp.float32),
      scratch_types=[
          pltpu.VMEM((p_rows, _L), jnp.float32),
          pltpu.VMEM((2, chunk), jnp.int32),
          pltpu.VMEM((2, chunk), jnp.float32),
          pltpu.SemaphoreType.DMA,
          pltpu.SemaphoreType.DMA,
          pltpu.SemaphoreType.DMA,
          pltpu.SemaphoreType.DMA,
      ],
  )
  def k(idx_hbm, params_hbm, out_hbm, params_v, idx_v, out_v,
        is0, is1, os0, os1):
    wid = lax.axis_index("s") * _NC + lax.axis_index("c")
    base = wid * n_per_w
    in_sems = (is0, is1)
    out_sems = (os0, os1)

    pltpu.sync_copy(params_hbm, params_v)
    acc = jnp.zeros((_L,), jnp.float32)
    for d in range(dim):
      acc = acc + params_v[d] * params_v[dim + d]
    s = 1.0 / (1.0 + jnp.exp(-(acc + params_v[2 * dim])))

    dnums = lax.GatherDimensionNumbers(
        offset_dims=(), collapsed_slice_dims=(0,), start_index_map=(0,))

    in_desc = [None, None]
    out_desc = [None, None]
    in_desc[0] = pltpu.async_copy(
        idx_hbm.at[pl.ds(base, chunk)], idx_v.at[0], in_sems[0])
    for c in range(n_chunks):
      cur = c % 2
      nxt = 1 - cur
      if c + 1 < n_chunks:
        in_desc[nxt] = pltpu.async_copy(
            idx_hbm.at[pl.ds(base + (c + 1) * chunk, chunk)],
            idx_v.at[nxt], in_sems[nxt])
      in_desc[cur].wait()
      if c >= 2:
        out_desc[cur].wait()
      src = idx_v.at[cur]
      dst = out_v.at[cur]

      @functools.partial(plsc.parallel_loop, 0, chunk // _L, unroll=8)
      def body(i):
        o = i * _L
        iv = src[pl.ds(o, _L)]
        dst[pl.ds(o, _L)] = lax.gather(
            s, iv[:, None], dnums, (1,),
            mode=lax.GatherScatterMode.PROMISE_IN_BOUNDS)

      out_desc[cur] = pltpu.async_copy(
          dst, out_hbm.at[pl.ds(base + c * chunk, chunk)], out_sems[cur])
    for d in range(min(2, n_chunks)):
      out_desc[(n_chunks - 1 - d) % 2].wait()

  return k(idx_flat, params)


def kernel(indices, table, W, b):
  n_vocab, dim = table.shape
  out_shape = indices.shape + (1,)
  n_total = indices.size

  idx_flat = indices.reshape(-1).astype(jnp.int32)
  tcols = jnp.zeros((dim, _L), jnp.float32).at[:, :n_vocab].set(
      table.astype(jnp.float32).T)
  wrows = jnp.broadcast_to(W.astype(jnp.float32).reshape(dim, 1), (dim, _L))
  brow = jnp.broadcast_to(b.astype(jnp.float32).reshape(1, 1), (1, _L))
  params = jnp.concatenate([tcols, wrows, brow], axis=0)

  assert n_total % _NW == 0
  n_per_w = n_total // _NW
  chunk = n_per_w
  while chunk * 16 > 420000:
    chunk //= 2
  assert n_per_w % chunk == 0 and chunk % _L == 0

  out_flat = _sc_lookup(idx_flat, params, n_total=n_total, dim=dim, chunk=chunk)
  return out_flat.reshape(out_shape)


# speedup vs baseline: 205.2372x; 1.6843x over previous
import functools

import jax
import jax.numpy as jnp
from jax import lax
from jax.experimental import pallas as pl
from jax.experimental.pallas import tpu as pltpu
from jax.experimental.pallas import tpu_sc as plsc

_NC = 2
_NS = 16
_NW = _NC * _NS
_L = 16


@functools.partial(jax.jit, static_argnames=("dim", "rblk"))
def _sc_lookup(idx, params, *, dim, rblk):
  n_rows, row_len = idx.shape
  rows_per_w = n_rows // _NW
  n_blks = rows_per_w // rblk
  p_rows = params.shape[0]

  col_offs = list(range(0, row_len - _L + 1, _L))
  if col_offs[-1] != row_len - _L:
    col_offs.append(row_len - _L)

  mesh = plsc.VectorSubcoreMesh(core_axis_name="c", subcore_axis_name="s")

  @functools.partial(
      pl.kernel,
      mesh=mesh,
      out_type=jax.ShapeDtypeStruct((n_rows, row_len), jnp.float32),
      scratch_types=[
          pltpu.VMEM((p_rows, _L), jnp.float32),
          pltpu.VMEM((2, rblk, row_len), jnp.int32),
          pltpu.VMEM((2, rblk, row_len), jnp.float32),
          pltpu.SemaphoreType.DMA,
          pltpu.SemaphoreType.DMA,
          pltpu.SemaphoreType.DMA,
          pltpu.SemaphoreType.DMA,
      ],
  )
  def k(idx_hbm, params_hbm, out_hbm, params_v, idx_v, out_v,
        is0, is1, os0, os1):
    wid = lax.axis_index("s") * _NC + lax.axis_index("c")
    base = wid * rows_per_w
    in_sems = (is0, is1)
    out_sems = (os0, os1)

    pltpu.sync_copy(params_hbm, params_v)
    acc = jnp.zeros((_L,), jnp.float32)
    for d in range(dim):
      acc = acc + params_v[d] * params_v[dim + d]
    s = 1.0 / (1.0 + jnp.exp(-(acc + params_v[2 * dim])))

    dnums = lax.GatherDimensionNumbers(
        offset_dims=(), collapsed_slice_dims=(0,), start_index_map=(0,))

    in_desc = [None, None]
    out_desc = [None, None]
    in_desc[0] = pltpu.async_copy(
        idx_hbm.at[pl.ds(base, rblk)], idx_v.at[0], in_sems[0])
    for c in range(n_blks):
      cur = c % 2
      nxt = 1 - cur
      if c + 1 < n_blks:
        in_desc[nxt] = pltpu.async_copy(
            idx_hbm.at[pl.ds(base + (c + 1) * rblk, rblk)],
            idx_v.at[nxt], in_sems[nxt])
      in_desc[cur].wait()
      if c >= 2:
        out_desc[cur].wait()
      src = idx_v.at[cur]
      dst = out_v.at[cur]

      @functools.partial(plsc.parallel_loop, 0, rblk, unroll=2)
      def body(r):
        for o in col_offs:
          iv = src[r, pl.ds(o, _L)]
          dst[r, pl.ds(o, _L)] = lax.gather(
              s, iv[:, None], dnums, (1,),
              mode=lax.GatherScatterMode.PROMISE_IN_BOUNDS)

      out_desc[cur] = pltpu.async_copy(
          dst, out_hbm.at[pl.ds(base + c * rblk, rblk)], out_sems[cur])
    for d in range(min(2, n_blks)):
      out_desc[(n_blks - 1 - d) % 2].wait()

  return k(idx, params)


def kernel(indices, table, W, b):
  n_vocab, dim = table.shape
  out_shape = indices.shape + (1,)
  n_rows, row_len = indices.shape

  idx = indices.astype(jnp.int32)
  tcols = jnp.zeros((dim, _L), jnp.float32).at[:, :n_vocab].set(
      table.astype(jnp.float32).T)
  wrows = jnp.broadcast_to(W.astype(jnp.float32).reshape(dim, 1), (dim, _L))
  brow = jnp.broadcast_to(b.astype(jnp.float32).reshape(1, 1), (1, _L))
  params = jnp.concatenate([tcols, wrows, brow], axis=0)

  assert n_rows % _NW == 0
  rows_per_w = n_rows // _NW
  padded_row = -(-row_len // 128) * 128
  rblk = rows_per_w
  while rblk * padded_row * 4 > 110000:
    rblk //= 2
  assert rows_per_w % rblk == 0 and rblk % 8 == 0

  out2d = _sc_lookup(idx, params, dim=dim, rblk=rblk)
  return out2d.reshape(out_shape)
